# baseline (speedup 1.0000x reference)
import jax
import jax.numpy as jnp
from jax import lax
from jax.experimental import pallas as pl
from jax.experimental.pallas import tpu as pltpu
from jax.experimental.pallas import tpu_sc as plsc

N_NODES = 10000
N_EDGES = 320000
D = 128

NC = 2
NS = 16
NW = NC * NS

C = 128
CH0 = 160
SUB = 4
CS = C // SUB
TOT_CHUNKS = NS * CH0
E_PAD = TOT_CHUNKS * C
N_PAD = 10240
RPT = N_PAD // NS
NBUF = 2
DEG_CHUNKS = TOT_CHUNKS // NW


def _deg_body(dst_hbm, zeros_hbm, out_hbm, didx_all, ones_v, deg_sh, sem):
    c = lax.axis_index("c")
    s = lax.axis_index("s")
    wid = s * NC + c

    z0 = s * RPT
    pltpu.sync_copy(zeros_hbm.at[pl.ds(z0, RPT)], deg_sh.at[pl.ds(z0, RPT)])

    @pl.loop(0, C // 16)
    def _(j):
        ones_v[pl.ds(j * 16, 16)] = jnp.ones((16,), jnp.float32)

    pltpu.sync_copy(dst_hbm.at[pl.ds(wid * DEG_CHUNKS, DEG_CHUNKS)], didx_all)
    plsc.subcore_barrier()

    @pl.loop(0, DEG_CHUNKS)
    def _(k):
        pltpu.sync_copy(ones_v, deg_sh.at[didx_all.at[k]], add=True)

    plsc.subcore_barrier()
    o0 = pl.multiple_of(c * N_PAD + z0, 8)
    pltpu.sync_copy(deg_sh.at[pl.ds(z0, RPT)], out_hbm.at[pl.ds(o0, RPT)])


def _agg_chunk_loop(h_tab, src_hbm, dst_hbm, acc_sh, sidx, didx, rows,
                    gsems, isems, dsems, e0, nch):
    for b in range(NBUF):
        off = pl.multiple_of(e0 + b * C, C)
        pltpu.async_copy(src_hbm.at[pl.ds(off, C)], sidx[b], isems[b])
        pltpu.async_copy(dst_hbm.at[pl.ds(off, C)], didx[b], dsems[b])
    for b in range(NBUF):
        pltpu.make_async_copy(src_hbm.at[pl.ds(0, C)], sidx[b], isems[b]).wait()
        for q in range(SUB):
            pltpu.async_copy(h_tab.at[sidx[b].at[pl.ds(q * CS, CS)]],
                             rows[b].at[pl.ds(q * CS, CS)], gsems[b])

    @pl.loop(0, nch - NBUF, step=NBUF)
    def _(j):
        for b in range(NBUF):
            k = j + b
            off = pl.multiple_of(e0 + (k + NBUF) * C, C)
            for q in range(SUB):
                pltpu.make_async_copy(h_tab.at[pl.ds(0, CS)],
                                      rows[b].at[pl.ds(q * CS, CS)],
                                      gsems[b]).wait()
            pltpu.async_copy(src_hbm.at[pl.ds(off, C)], sidx[b], isems[b])
            pltpu.make_async_copy(src_hbm.at[pl.ds(0, C)], didx[b], dsems[b]).wait()
            pltpu.sync_copy(rows[b], acc_sh.at[didx[b]], add=True)
            pltpu.async_copy(dst_hbm.at[pl.ds(off, C)], didx[b], dsems[b])
            pltpu.make_async_copy(src_hbm.at[pl.ds(0, C)], sidx[b], isems[b]).wait()
            for q in range(SUB):
                pltpu.async_copy(h_tab.at[sidx[b].at[pl.ds(q * CS, CS)]],
                                 rows[b].at[pl.ds(q * CS, CS)], gsems[b])

    for b in range(NBUF):
        for q in range(SUB):
            pltpu.make_async_copy(h_tab.at[pl.ds(0, CS)],
                                  rows[b].at[pl.ds(q * CS, CS)], gsems[b]).wait()
        pltpu.make_async_copy(src_hbm.at[pl.ds(0, C)], didx[b], dsems[b]).wait()
        pltpu.sync_copy(rows[b], acc_sh.at[didx[b]], add=True)


def _agg_body(h_hbm, src_hbm, dst_hbm, zrows_hbm, out_hbm,
              sidx0, sidx1, didx0, didx1, rows0, rows1,
              acc_sh, gsem0, gsem1, isem0, isem1, dsem0, dsem1):
    c = lax.axis_index("c")
    s = lax.axis_index("s")
    sidx = (sidx0, sidx1)
    didx = (didx0, didx1)
    rows = (rows0, rows1)
    gsems = (gsem0, gsem1)
    isems = (isem0, isem1)
    dsems = (dsem0, dsem1)

    z0 = s * RPT
    pltpu.sync_copy(zrows_hbm.at[pl.ds(z0, RPT)], acc_sh.at[pl.ds(z0, RPT)])
    plsc.subcore_barrier()

    @pl.when(c == 0)
    def _():
        _agg_chunk_loop(h_hbm, src_hbm, dst_hbm, acc_sh, sidx, didx, rows,
                        gsems, isems, dsems, s * CH0 * C, CH0)

    plsc.subcore_barrier()
    pltpu.sync_copy(acc_sh.at[pl.ds(z0, RPT)], out_hbm.at[c, pl.ds(z0, RPT)])


def _h_body(feat_ref, d_ref, o_ref):
    deg = d_ref[:, 0:1] + d_ref[:, 1:2]
    norm = jax.lax.rsqrt(jnp.clip(deg, 1.0, None))
    o_ref[...] = feat_ref[...] * norm


def _out_body(p0_ref, p1_ref, w_ref, d_ref, b_ref, o_ref):
    acc = p0_ref[...] + p1_ref[...]
    deg = d_ref[:, 0:1] + d_ref[:, 1:2]
    norm = jax.lax.rsqrt(jnp.clip(deg, 1.0, None))
    r = jnp.dot(acc, w_ref[...], preferred_element_type=jnp.float32)
    o_ref[...] = r * norm + b_ref[...]


def kernel(feat, edge_index, weight, bias):
    src = edge_index[0].astype(jnp.int32)
    dst = edge_index[1].astype(jnp.int32)

    pad = E_PAD - N_EDGES
    src_p = jnp.concatenate([src, jnp.zeros((pad,), jnp.int32)])
    dst_p = jnp.concatenate([dst, jnp.full((pad,), N_NODES, jnp.int32)])

    zeros1 = jnp.zeros((N_PAD,), jnp.float32)
    zeros2 = jnp.zeros((N_PAD, D), jnp.float32)

    mesh = plsc.VectorSubcoreMesh(core_axis_name="c", subcore_axis_name="s")

    deg_k = pl.kernel(
        _deg_body,
        out_type=jax.ShapeDtypeStruct((NC * N_PAD,), jnp.float32),
        mesh=mesh,
        scratch_types=[
            pltpu.VMEM((DEG_CHUNKS, C), jnp.int32),
            pltpu.VMEM((C,), jnp.float32),
            pltpu.VMEM_SHARED((N_PAD,), jnp.float32),
            pltpu.SemaphoreType.DMA,
        ],
    )
    deg2 = deg_k(dst_p.reshape(TOT_CHUNKS, C), zeros1).reshape(NC, N_PAD)
    deg_t = deg2.T[:N_NODES]

    h = pl.pallas_call(
        _h_body,
        out_shape=jax.ShapeDtypeStruct((N_NODES, D), jnp.float32),
    )(feat, deg_t)

    agg_k = pl.kernel(
        _agg_body,
        out_type=jax.ShapeDtypeStruct((NC, N_PAD, D), jnp.float32),
        mesh=mesh,
        scratch_types=(
            [pltpu.VMEM((C,), jnp.int32)] * (2 * NBUF)
            + [pltpu.VMEM((C, D), jnp.float32)] * NBUF
            + [pltpu.VMEM_SHARED((N_PAD, D), jnp.float32)]
            + [pltpu.SemaphoreType.DMA] * (3 * NBUF)
        ),
    )
    p2 = agg_k(h, src_p, dst_p, zeros2)

    out = pl.pallas_call(
        _out_body,
        out_shape=jax.ShapeDtypeStruct((N_NODES, D), jnp.float32),
    )(p2[0, :N_NODES], p2[1, :N_NODES], weight, deg_t, bias.reshape(1, D))
    return out


# speedup vs baseline: 1.3007x; 1.3007x over previous
import jax
import jax.numpy as jnp
from jax import lax
from jax.experimental import pallas as pl
from jax.experimental.pallas import tpu as pltpu
from jax.experimental.pallas import tpu_sc as plsc

N_NODES = 10000
N_EDGES = 320000
D = 128

NC = 2
NS = 16
NW = NC * NS

C = 128
CH0 = 120
CH1 = 40
TOT_CHUNKS = NS * (CH0 + CH1)
E_PAD = TOT_CHUNKS * C
N_PAD = 10240
RPT = N_PAD // NS
NBUF = 2
DEG_CHUNKS = TOT_CHUNKS // NW


def _deg_body(dst_hbm, zeros_hbm, out_hbm, didx_all, ones_v, deg_sh, sem):
    c = lax.axis_index("c")
    s = lax.axis_index("s")
    wid = s * NC + c

    z0 = s * RPT
    pltpu.sync_copy(zeros_hbm.at[pl.ds(z0, RPT)], deg_sh.at[pl.ds(z0, RPT)])

    @pl.loop(0, C // 16)
    def _(j):
        ones_v[pl.ds(j * 16, 16)] = jnp.ones((16,), jnp.float32)

    pltpu.sync_copy(dst_hbm.at[pl.ds(wid * DEG_CHUNKS, DEG_CHUNKS)], didx_all)
    plsc.subcore_barrier()

    @pl.loop(0, DEG_CHUNKS)
    def _(k):
        pltpu.sync_copy(ones_v, deg_sh.at[didx_all.at[k]], add=True)

    plsc.subcore_barrier()
    o0 = pl.multiple_of(c * N_PAD + z0, 8)
    pltpu.sync_copy(deg_sh.at[pl.ds(z0, RPT)], out_hbm.at[pl.ds(o0, RPT)])


def _agg_chunk_loop(h_tab, src_hbm, acc_sh, didx_all, sidx, rows, gsems, isems,
                    e0, nch):
    for b in range(NBUF):
        pltpu.async_copy(src_hbm.at[pl.ds(pl.multiple_of(e0 + b * C, C), C)],
                         sidx[b], isems[b])
    for b in range(NBUF):
        pltpu.make_async_copy(src_hbm.at[pl.ds(0, C)], sidx[b], isems[b]).wait()
        pltpu.async_copy(h_tab.at[sidx[b]], rows[b], gsems[b])

    @pl.loop(0, nch - NBUF, step=NBUF)
    def _(j):
        for b in range(NBUF):
            k = j + b
            pltpu.make_async_copy(h_tab.at[pl.ds(0, C)], rows[b], gsems[b]).wait()
            pltpu.async_copy(
                src_hbm.at[pl.ds(pl.multiple_of(e0 + (k + NBUF) * C, C), C)],
                sidx[b], isems[b])
            pltpu.sync_copy(rows[b], acc_sh.at[didx_all.at[k]], add=True)
            pltpu.make_async_copy(src_hbm.at[pl.ds(0, C)], sidx[b], isems[b]).wait()
            pltpu.async_copy(h_tab.at[sidx[b]], rows[b], gsems[b])

    for b in range(NBUF):
        k = nch - NBUF + b
        pltpu.make_async_copy(h_tab.at[pl.ds(0, C)], rows[b], gsems[b]).wait()
        pltpu.sync_copy(rows[b], acc_sh.at[didx_all.at[k]], add=True)


def _agg_body(h_hbm, src_hbm, dst_hbm, zrows_hbm, out_hbm,
              didx_all, sidx0, sidx1, rows0, rows1,
              acc_sh, gsem0, gsem1, isem0, isem1):
    c = lax.axis_index("c")
    s = lax.axis_index("s")
    sidx = (sidx0, sidx1)
    rows = (rows0, rows1)
    gsems = (gsem0, gsem1)
    isems = (isem0, isem1)

    z0 = s * RPT
    pltpu.sync_copy(zrows_hbm.at[pl.ds(z0, RPT)], acc_sh.at[pl.ds(z0, RPT)])

    @pl.when(c == 0)
    def _():
        pltpu.sync_copy(dst_hbm.at[pl.ds(s * CH0, CH0)],
                        didx_all.at[pl.ds(0, CH0)])

    @pl.when(c == 1)
    def _():
        pltpu.sync_copy(dst_hbm.at[pl.ds(NS * CH0 + s * CH1, CH1)],
                        didx_all.at[pl.ds(0, CH1)])

    plsc.subcore_barrier()

    @pl.when(c == 0)
    def _():
        _agg_chunk_loop(h_hbm.at[0], src_hbm, acc_sh, didx_all, sidx, rows,
                        gsems, isems, s * CH0 * C, CH0)

    @pl.when(c == 1)
    def _():
        _agg_chunk_loop(h_hbm.at[1], src_hbm, acc_sh, didx_all, sidx, rows,
                        gsems, isems, (NS * CH0 + s * CH1) * C, CH1)

    plsc.subcore_barrier()
    pltpu.sync_copy(acc_sh.at[pl.ds(z0, RPT)], out_hbm.at[c, pl.ds(z0, RPT)])


def _h_body(feat_ref, d_ref, o_ref):
    deg = d_ref[:, 0:1] + d_ref[:, 1:2]
    norm = jax.lax.rsqrt(jnp.clip(deg, 1.0, None))
    hv = feat_ref[...] * norm
    o_ref[0] = hv
    o_ref[1] = hv


def _out_body(p0_ref, p1_ref, w_ref, d_ref, b_ref, o_ref):
    acc = p0_ref[...] + p1_ref[...]
    deg = d_ref[:, 0:1] + d_ref[:, 1:2]
    norm = jax.lax.rsqrt(jnp.clip(deg, 1.0, None))
    r = jnp.dot(acc, w_ref[...], preferred_element_type=jnp.float32)
    o_ref[...] = r * norm + b_ref[...]


def kernel(feat, edge_index, weight, bias):
    src = edge_index[0].astype(jnp.int32)
    dst = edge_index[1].astype(jnp.int32)

    pad = E_PAD - N_EDGES
    src_p = jnp.concatenate([src, jnp.zeros((pad,), jnp.int32)])
    dst_p = jnp.concatenate([dst, jnp.full((pad,), N_NODES, jnp.int32)])
    dst2d = dst_p.reshape(TOT_CHUNKS, C)

    zeros1 = jnp.zeros((N_PAD,), jnp.float32)
    zeros2 = jnp.zeros((N_PAD, D), jnp.float32)

    mesh = plsc.VectorSubcoreMesh(core_axis_name="c", subcore_axis_name="s")

    deg_k = pl.kernel(
        _deg_body,
        out_type=jax.ShapeDtypeStruct((NC * N_PAD,), jnp.float32),
        mesh=mesh,
        scratch_types=[
            pltpu.VMEM((DEG_CHUNKS, C), jnp.int32),
            pltpu.VMEM((C,), jnp.float32),
            pltpu.VMEM_SHARED((N_PAD,), jnp.float32),
            pltpu.SemaphoreType.DMA,
        ],
    )
    deg2 = deg_k(dst2d, zeros1).reshape(NC, N_PAD)
    deg_t = deg2.T[:N_NODES]

    h = pl.pallas_call(
        _h_body,
        out_shape=jax.ShapeDtypeStruct((NC, N_NODES, D), jnp.float32),
    )(feat, deg_t)

    agg_k = pl.kernel(
        _agg_body,
        out_type=jax.ShapeDtypeStruct((NC, N_PAD, D), jnp.float32),
        mesh=mesh,
        scratch_types=(
            [pltpu.VMEM((CH0, C), jnp.int32)]
            + [pltpu.VMEM((C,), jnp.int32)] * NBUF
            + [pltpu.VMEM((C, D), jnp.float32)] * NBUF
            + [pltpu.VMEM_SHARED((N_PAD, D), jnp.float32)]
            + [pltpu.SemaphoreType.DMA] * (2 * NBUF)
        ),
    )
    p2 = agg_k(h, src_p, dst2d, zeros2)

    out = pl.pallas_call(
        _out_body,
        out_shape=jax.ShapeDtypeStruct((N_NODES, D), jnp.float32),
    )(p2[0, :N_NODES], p2[1, :N_NODES], weight, deg_t, bias.reshape(1, D))
    return out
